# Initial kernel scaffold; baseline (speedup 1.0000x reference)
import jax
import jax.numpy as jnp
from jax.experimental import pallas as pl


def kernel(x, edge_index, edge_attr, W0, b0, W1, b1, W2, b2, W3, b3):
    raise NotImplementedError("write your pallas kernel here")


# speedup vs baseline: 9.9923x; 9.9923x over previous
import functools

import jax
import jax.numpy as jnp
from jax import lax
from jax.experimental import pallas as pl
from jax.experimental.pallas import tpu as pltpu
from jax.experimental.pallas import tpu_sc as plsc

NC = 2
NS = 16
NW = NC * NS

_F32 = jnp.float32


def _sc_mesh():
    return plsc.VectorSubcoreMesh(core_axis_name="c", subcore_axis_name="s")


@functools.cache
def _deg_kernel(E, N):
    C = 80
    EPW = E // NW
    assert E % NW == 0 and EPW % C == 0
    ZR = ((N + NS - 1) // NS + 127) // 128 * 128
    NP = NS * ZR

    @functools.partial(
        pl.kernel,
        out_type=jax.ShapeDtypeStruct((NC, NP), _F32),
        mesh=_sc_mesh(),
        scratch_types=[
            pltpu.VMEM((C,), jnp.int32),
            pltpu.VMEM((C,), _F32),
            pltpu.VMEM((ZR,), _F32),
            pltpu.VMEM_SHARED((NP,), _F32),
        ],
    )
    def deg_kernel(dst_hbm, ew_hbm, out_hbm, didx_v, ew_v, zb_v, deg_sh):
        cid = lax.axis_index("c")
        sid = lax.axis_index("s")
        wid = cid * NS + sid

        def zbody(i, _):
            zb_v[pl.ds(i * 16, 16)] = jnp.zeros((16,), _F32)
            return 0
        lax.fori_loop(0, ZR // 16, zbody, 0)
        r0 = pl.multiple_of(sid * ZR, 8)
        pltpu.sync_copy(zb_v, deg_sh.at[pl.ds(r0, ZR)])
        plsc.subcore_barrier()

        base0 = wid * EPW

        def chunk(k, _):
            base = pl.multiple_of(base0 + k * C, 8)
            pltpu.sync_copy(dst_hbm.at[pl.ds(base, C)], didx_v)
            pltpu.sync_copy(ew_hbm.at[pl.ds(base, C)], ew_v)
            pltpu.sync_copy(ew_v, deg_sh.at[didx_v], add=True)
            return 0
        lax.fori_loop(0, EPW // C, chunk, 0)
        plsc.subcore_barrier()
        pltpu.sync_copy(deg_sh.at[pl.ds(r0, ZR)], out_hbm.at[cid, pl.ds(r0, ZR)])

    return deg_kernel


@functools.cache
def _msg_kernel(E, N, D):
    C = 80
    EPW = E // NW
    assert E % NW == 0 and EPW % C == 0 and D % 16 == 0
    ZR = ((N + NS - 1) // NS + 127) // 128 * 128
    NP = NS * ZR
    ZCOPIES = ZR // C
    assert ZR % C == 0

    @functools.partial(
        pl.kernel,
        out_type=jax.ShapeDtypeStruct((NC, NP, D), _F32),
        mesh=_sc_mesh(),
        scratch_types=[
            pltpu.VMEM((C,), jnp.int32),
            pltpu.VMEM((C,), jnp.int32),
            pltpu.VMEM((C,), _F32),
            pltpu.VMEM((C, D), _F32),
            pltpu.VMEM_SHARED((NP, D), _F32),
            pltpu.SemaphoreType.DMA,
        ],
    )
    def msg_kernel(hs_hbm, src_hbm, dst_hbm, ew_hbm, out_hbm,
                   sidx_v, didx_v, ew_v, rows_v, msg_sh, sem):
        cid = lax.axis_index("c")
        sid = lax.axis_index("s")
        wid = cid * NS + sid

        def zrow(i, _):
            for j in range(D // 16):
                rows_v[i, pl.ds(j * 16, 16)] = jnp.zeros((16,), _F32)
            return 0
        lax.fori_loop(0, C, zrow, 0)
        r0 = sid * ZR
        for t in range(ZCOPIES):
            pltpu.sync_copy(rows_v, msg_sh.at[pl.ds(r0 + t * C, C)])
        plsc.subcore_barrier()

        base0 = wid * EPW

        def chunk(k, _):
            base = pl.multiple_of(base0 + k * C, 8)
            pltpu.sync_copy(src_hbm.at[pl.ds(base, C)], sidx_v)
            cp = pltpu.async_copy(hs_hbm.at[sidx_v], rows_v, sem)
            pltpu.sync_copy(dst_hbm.at[pl.ds(base, C)], didx_v)
            pltpu.sync_copy(ew_hbm.at[pl.ds(base, C)], ew_v)
            cp.wait()

            def scale(g, _):
                vew = ew_v[pl.ds(g * 16, 16)]
                for l in range(16):
                    w = vew[l]
                    row = g * 16 + l
                    for j in range(D // 16):
                        sl = pl.ds(j * 16, 16)
                        rows_v[row, sl] = rows_v[row, sl] * w
                return 0
            lax.fori_loop(0, C // 16, scale, 0)
            pltpu.sync_copy(rows_v, msg_sh.at[didx_v], add=True)
            return 0
        lax.fori_loop(0, EPW // C, chunk, 0)
        plsc.subcore_barrier()
        pltpu.sync_copy(msg_sh.at[pl.ds(r0, ZR)],
                        out_hbm.at[cid, pl.ds(r0, ZR)])

    return msg_kernel


def _dinv_tc(degp, NP):
    def body(dp_ref, o_ref):
        d = dp_ref[0:1, :] + dp_ref[1:2, :] + 1.0
        o_ref[...] = jnp.where(d > 0, lax.rsqrt(jnp.where(d > 0, d, 1.0)), 0.0)

    return pl.pallas_call(
        body, out_shape=jax.ShapeDtypeStruct((1, NP), _F32))(degp)


_BR = 1000


def _layer0_tc(x, W, dinv):
    N, D = x.shape
    G = N // _BR

    def body(x_ref, w_ref, dv_ref, o_ref):
        h = jnp.dot(x_ref[...], w_ref[...], preferred_element_type=_F32)
        o_ref[...] = h * dv_ref[...]

    return pl.pallas_call(
        body,
        grid=(G,),
        in_specs=[
            pl.BlockSpec((_BR, D), lambda i: (i, 0)),
            pl.BlockSpec((D, D), lambda i: (0, 0)),
            pl.BlockSpec((_BR, 1), lambda i: (i, 0)),
        ],
        out_specs=pl.BlockSpec((_BR, D), lambda i: (i, 0)),
        out_shape=jax.ShapeDtypeStruct((N, D), _F32),
    )(x, W, dinv)


def _layer_mid_tc(p, hs, dinv, b, W):
    N, D = hs.shape
    G = N // _BR

    def body(p_ref, hs_ref, dv_ref, b_ref, w_ref, o_ref):
        conv = (p_ref[0] + p_ref[1] + hs_ref[...]) * dv_ref[...] + b_ref[...]
        h = jnp.maximum(conv, 0.0)
        o_ref[...] = jnp.dot(h, w_ref[...], preferred_element_type=_F32) * dv_ref[...]

    return pl.pallas_call(
        body,
        grid=(G,),
        in_specs=[
            pl.BlockSpec((NC, _BR, D), lambda i: (0, i, 0)),
            pl.BlockSpec((_BR, D), lambda i: (i, 0)),
            pl.BlockSpec((_BR, 1), lambda i: (i, 0)),
            pl.BlockSpec((1, D), lambda i: (0, 0)),
            pl.BlockSpec((D, D), lambda i: (0, 0)),
        ],
        out_specs=pl.BlockSpec((_BR, D), lambda i: (i, 0)),
        out_shape=jax.ShapeDtypeStruct((N, D), _F32),
    )(p, hs, dinv, b, W)


def _layer_dual_tc(p, hs, dinv, b, W2, W3):
    N, D = hs.shape
    G = N // _BR

    def body(p_ref, hs_ref, dv_ref, b_ref, w2_ref, w3_ref, o2_ref, o3_ref):
        conv = (p_ref[0] + p_ref[1] + hs_ref[...]) * dv_ref[...] + b_ref[...]
        h = jnp.maximum(conv, 0.0)
        o2_ref[...] = jnp.dot(h, w2_ref[...], preferred_element_type=_F32) * dv_ref[...]
        o3_ref[...] = jnp.dot(h, w3_ref[...], preferred_element_type=_F32) * dv_ref[...]

    return pl.pallas_call(
        body,
        grid=(G,),
        in_specs=[
            pl.BlockSpec((NC, _BR, D), lambda i: (0, i, 0)),
            pl.BlockSpec((_BR, D), lambda i: (i, 0)),
            pl.BlockSpec((_BR, 1), lambda i: (i, 0)),
            pl.BlockSpec((1, D), lambda i: (0, 0)),
            pl.BlockSpec((D, D), lambda i: (0, 0)),
            pl.BlockSpec((D, D), lambda i: (0, 0)),
        ],
        out_specs=[
            pl.BlockSpec((_BR, D), lambda i: (i, 0)),
            pl.BlockSpec((_BR, D), lambda i: (i, 0)),
        ],
        out_shape=[
            jax.ShapeDtypeStruct((N, D), _F32),
            jax.ShapeDtypeStruct((N, D), _F32),
        ],
    )(p, hs, dinv, b, W2, W3)


def _final_tc(p2, hs2, b2, p3, hs3, b3, dinv):
    N, D = hs2.shape
    G = N // _BR

    def body(p2_ref, hs2_ref, b2_ref, p3_ref, hs3_ref, b3_ref, dv_ref,
             mu_ref, lv_ref):
        dv = dv_ref[...]
        mu_ref[...] = (p2_ref[0] + p2_ref[1] + hs2_ref[...]) * dv + b2_ref[...]
        z = (p3_ref[0] + p3_ref[1] + hs3_ref[...]) * dv + b3_ref[...]
        lv_ref[...] = jnp.maximum(z, 0.0) + jnp.log1p(jnp.exp(-jnp.abs(z)))

    return pl.pallas_call(
        body,
        grid=(G,),
        in_specs=[
            pl.BlockSpec((NC, _BR, D), lambda i: (0, i, 0)),
            pl.BlockSpec((_BR, D), lambda i: (i, 0)),
            pl.BlockSpec((1, D), lambda i: (0, 0)),
            pl.BlockSpec((NC, _BR, D), lambda i: (0, i, 0)),
            pl.BlockSpec((_BR, D), lambda i: (i, 0)),
            pl.BlockSpec((1, D), lambda i: (0, 0)),
            pl.BlockSpec((_BR, 1), lambda i: (i, 0)),
        ],
        out_specs=[
            pl.BlockSpec((_BR, D), lambda i: (i, 0)),
            pl.BlockSpec((_BR, D), lambda i: (i, 0)),
        ],
        out_shape=[
            jax.ShapeDtypeStruct((N, D), _F32),
            jax.ShapeDtypeStruct((N, D), _F32),
        ],
    )(p2, hs2, b2, p3, hs3, b3, dinv)


def kernel(x, edge_index, edge_attr, W0, b0, W1, b1, W2, b2, W3, b3):
    N, D = x.shape
    E = edge_attr.shape[0]
    src = edge_index[0]
    dst = edge_index[1]
    b0r = b0.reshape(1, D)
    b1r = b1.reshape(1, D)
    b2r = b2.reshape(1, D)
    b3r = b3.reshape(1, D)

    degp = _deg_kernel(E, N)(dst, edge_attr)
    NP = degp.shape[1]
    dinv = _dinv_tc(degp, NP).reshape(NP, 1)

    msg = _msg_kernel(E, N, D)

    hs0 = _layer0_tc(x, W0, dinv)
    p0 = msg(hs0, src, dst, edge_attr)
    hs1 = _layer_mid_tc(p0, hs0, dinv, b0r, W1)
    p1 = msg(hs1, src, dst, edge_attr)
    hs2, hs3 = _layer_dual_tc(p1, hs1, dinv, b1r, W2, W3)
    p2 = msg(hs2, src, dst, edge_attr)
    p3 = msg(hs3, src, dst, edge_attr)
    mu, logvar = _final_tc(p2, hs2, b2r, p3, hs3, b3r, dinv)
    return (mu, logvar)
